# baseline (speedup 1.0000x reference)
import functools

import jax
import jax.numpy as jnp
from jax import lax
from jax.experimental import pallas as pl
from jax.experimental.pallas import tpu as pltpu
from jax.experimental.pallas import tpu_sc as plsc

NC = 2
NS = 16
NW = NC * NS
L = 16

BATCH = 16384
D = 128
CB = 128
BPW = BATCH // NW
CH = BPW // CB
GROUPS = D // L


def _sc_partials(x, labels, centers):
  mesh = plsc.VectorSubcoreMesh(core_axis_name="c", subcore_axis_name="s")

  @functools.partial(
      pl.kernel,
      out_type=jax.ShapeDtypeStruct((NW, L), jnp.float32),
      mesh=mesh,
      scratch_types=[
          pltpu.VMEM((BPW,), jnp.int32),
          pltpu.VMEM((CB, D), jnp.float32),
          pltpu.VMEM((CB, D), jnp.float32),
          pltpu.VMEM((CB, D), jnp.float32),
          pltpu.VMEM((CB, D), jnp.float32),
          pltpu.VMEM((2 * L * L,), jnp.float32),
          pltpu.VMEM((L,), jnp.float32),
          pltpu.SemaphoreType.DMA,
          pltpu.SemaphoreType.DMA,
          pltpu.SemaphoreType.DMA,
          pltpu.SemaphoreType.DMA,
      ],
      compiler_params=pltpu.CompilerParams(needs_layout_passes=False),
  )
  def sc_kernel(x_hbm, lab_hbm, cen_hbm, out_hbm, idx_v, x0, x1, c0, c1,
                rs_buf, acc_v, semx0, semx1, semc0, semc1):
    wid = lax.axis_index("s") * NC + lax.axis_index("c")
    base = wid * BPW
    xbufs = [x0, x1]
    cbufs = [c0, c1]
    semx = [semx0, semx1]
    semc = [semc0, semc1]

    pltpu.sync_copy(lab_hbm.at[pl.ds(base, CB)], idx_v.at[pl.ds(0, CB)])

    def start(kk):
      sl = kk % 2
      hx = pltpu.async_copy(x_hbm.at[pl.ds(base + kk * CB, CB)],
                            xbufs[sl], semx[sl])
      hc = pltpu.async_copy(cen_hbm.at[idx_v.at[pl.ds(kk * CB, CB)]],
                            cbufs[sl], semc[sl])
      return hx, hc

    rows16 = jnp.arange(L, dtype=jnp.int32) * L

    def _tree_sum(vs):
      while len(vs) > 1:
        vs = [vs[2 * i] + vs[2 * i + 1] for i in range(len(vs) // 2)] + (
            [vs[-1]] if len(vs) % 2 else [])
      return vs[0]

    def chunk_compute(acc, sl):
      xb = xbufs[sl]
      cb = cbufs[sl]

      def one_block(b, acc, half):
        rbase = half * (L * L)
        s0 = b * L
        for i in range(L):
          s = s0 + i
          sqs = []
          for g in range(GROUPS):
            dv = xb[s, pl.ds(g * L, L)] - cb[s, pl.ds(g * L, L)]
            sqs.append(dv * dv)
          rs_buf[pl.ds(rbase + i * L, L)] = _tree_sum(sqs)
        cols = [plsc.load_gather(rs_buf, [rows16 + (rbase + col)])
                for col in range(L)]
        dist = _tree_sum(cols)
        dist = jnp.minimum(jnp.maximum(dist, 1e-12), 1e12)
        return acc + dist

      def blk_pair(j, acc):
        acc = one_block(2 * j, acc, 0)
        acc = one_block(2 * j + 1, acc, 1)
        return acc

      return lax.fori_loop(0, CB // L // 2, blk_pair, acc)

    handles = start(0)
    pltpu.sync_copy(lab_hbm.at[pl.ds(base + CB, BPW - CB)],
                    idx_v.at[pl.ds(CB, BPW - CB)])
    acc = jnp.zeros((L,), jnp.float32)
    for kk in range(CH):
      hx, hc = handles
      if kk + 1 < CH:
        handles = start(kk + 1)
      hx.wait()
      hc.wait()
      acc = chunk_compute(acc, kk % 2)

    acc_v[...] = acc
    pltpu.sync_copy(acc_v, out_hbm.at[wid])

  return sc_kernel(x, labels, centers)


def _final_mean(partials):
  def body(p_ref, o_ref):
    o_ref[...] = jnp.sum(p_ref[...]).reshape(1, 1) * (1.0 / BATCH)

  return pl.pallas_call(
      body,
      out_shape=jax.ShapeDtypeStruct((1, 1), jnp.float32),
  )(partials)


def kernel(x, labels, centers):
  partials = _sc_partials(x, labels.astype(jnp.int32), centers)
  return _final_mean(partials)[0, 0]


# speedup vs baseline: 1.0393x; 1.0393x over previous
import functools

import jax
import jax.numpy as jnp
from jax import lax
from jax.experimental import pallas as pl
from jax.experimental.pallas import tpu as pltpu
from jax.experimental.pallas import tpu_sc as plsc

NC = 2
NS = 16
NW = NC * NS
L = 16

BATCH = 16384
D = 128
CB = 64
BPW = BATCH // NW
CH = BPW // CB
GROUPS = D // L
UNROLL = 8


def _sc_partials(x, labels, centers):
  mesh = plsc.VectorSubcoreMesh(core_axis_name="c", subcore_axis_name="s")

  @functools.partial(
      pl.kernel,
      out_type=jax.ShapeDtypeStruct((NW, L), jnp.float32),
      mesh=mesh,
      scratch_types=[
          pltpu.VMEM((BPW,), jnp.int32),
          pltpu.VMEM((CB, D), jnp.float32),
          pltpu.VMEM((CB, D), jnp.float32),
          pltpu.VMEM((CB, D), jnp.float32),
          pltpu.VMEM((CB, D), jnp.float32),
          pltpu.VMEM((L,), jnp.float32),
          pltpu.SemaphoreType.DMA,
          pltpu.SemaphoreType.DMA,
          pltpu.SemaphoreType.DMA,
          pltpu.SemaphoreType.DMA,
      ],
      compiler_params=pltpu.CompilerParams(needs_layout_passes=False),
  )
  def sc_kernel(x_hbm, lab_hbm, cen_hbm, out_hbm, idx_v, x0, x1, c0, c1,
                acc_v, semx0, semx1, semc0, semc1):
    wid = lax.axis_index("s") * NC + lax.axis_index("c")
    base = wid * BPW
    xbufs = [x0, x1]
    cbufs = [c0, c1]
    semx = [semx0, semx1]
    semc = [semc0, semc1]

    def start(kk):
      sl = kk % 2
      hx = pltpu.async_copy(x_hbm.at[pl.ds(base + kk * CB, CB)],
                            xbufs[sl], semx[sl])
      hc = pltpu.async_copy(cen_hbm.at[idx_v.at[pl.ds(kk * CB, CB)]],
                            cbufs[sl], semc[sl])
      return hx, hc

    pltpu.sync_copy(lab_hbm.at[pl.ds(base, 2 * CB)], idx_v.at[pl.ds(0, 2 * CB)])
    handles = [start(0), start(1)]
    pltpu.sync_copy(lab_hbm.at[pl.ds(base + 2 * CB, BPW - 2 * CB)],
                    idx_v.at[pl.ds(2 * CB, BPW - 2 * CB)])

    def chunk_compute(acc, sl):
      xb = xbufs[sl]
      cb = cbufs[sl]

      def blk(b, acc):
        s0 = b * UNROLL
        tots = []
        for i in range(UNROLL):
          s = s0 + i
          r0 = None
          r1 = None
          for g in range(0, GROUPS, 2):
            dv0 = xb[s, pl.ds(g * L, L)] - cb[s, pl.ds(g * L, L)]
            dv1 = xb[s, pl.ds((g + 1) * L, L)] - cb[s, pl.ds((g + 1) * L, L)]
            sq0 = dv0 * dv0
            sq1 = dv1 * dv1
            r0 = sq0 if r0 is None else r0 + sq0
            r1 = sq1 if r1 is None else r1 + sq1
          tot = jnp.sum(r0 + r1)
          tots.append(jnp.minimum(jnp.maximum(tot, 1e-12), 1e12))
        return acc + ((tots[0] + tots[1]) + (tots[2] + tots[3])) + (
            (tots[4] + tots[5]) + (tots[6] + tots[7]))

      return lax.fori_loop(0, CB // UNROLL, blk, acc)

    acc = jnp.float32(0.0)
    for kk in range(CH):
      sl = kk % 2
      hx, hc = handles[sl]
      hx.wait()
      hc.wait()
      acc = chunk_compute(acc, sl)
      if kk + 2 < CH:
        handles[sl] = start(kk + 2)

    lane = jnp.arange(L, dtype=jnp.int32)
    acc_v[...] = jnp.where(lane == 0, acc, jnp.float32(0.0))
    pltpu.sync_copy(acc_v, out_hbm.at[wid])

  return sc_kernel(x, labels, centers)


def _final_mean(partials):
  def body(p_ref, o_ref):
    o_ref[...] = jnp.sum(p_ref[...]).reshape(1, 1) * (1.0 / BATCH)

  return pl.pallas_call(
      body,
      out_shape=jax.ShapeDtypeStruct((1, 1), jnp.float32),
  )(partials)


def kernel(x, labels, centers):
  partials = _sc_partials(x, labels.astype(jnp.int32), centers)
  return _final_mean(partials)[0, 0]


# speedup vs baseline: 1.1335x; 1.0907x over previous
import functools

import jax
import jax.numpy as jnp
from jax import lax
from jax.experimental import pallas as pl
from jax.experimental.pallas import tpu as pltpu
from jax.experimental.pallas import tpu_sc as plsc

NC = 2
NS = 16
NW = NC * NS
L = 16

BATCH = 16384
D = 128
CB = 128
BPW = BATCH // NW
CH = BPW // CB
GROUPS = D // L


def _sc_partials(x, labels, centers):
  mesh = plsc.VectorSubcoreMesh(core_axis_name="c", subcore_axis_name="s")

  @functools.partial(
      pl.kernel,
      out_type=jax.ShapeDtypeStruct((NW, L), jnp.float32),
      mesh=mesh,
      scratch_types=[
          pltpu.VMEM((BPW,), jnp.int32),
          pltpu.VMEM((BPW, D), jnp.float32),
          pltpu.VMEM((CB, D), jnp.float32),
          pltpu.VMEM((CB, D), jnp.float32),
          pltpu.VMEM((L * L,), jnp.float32),
          pltpu.VMEM((L,), jnp.float32),
          pltpu.SemaphoreType.DMA,
          pltpu.SemaphoreType.DMA,
          pltpu.SemaphoreType.DMA,
          pltpu.SemaphoreType.DMA,
      ],
      compiler_params=pltpu.CompilerParams(needs_layout_passes=False),
  )
  def sc_kernel(x_hbm, lab_hbm, cen_hbm, out_hbm, idx_v, x_full, c0, c1,
                rs_buf, acc_v, semx0, semx1, semc0, semc1):
    wid = lax.axis_index("s") * NC + lax.axis_index("c")
    base = wid * BPW
    cbufs = [c0, c1]
    semc = [semc0, semc1]
    half = BPW // 2

    hx0 = pltpu.async_copy(x_hbm.at[pl.ds(base, half)],
                           x_full.at[pl.ds(0, half)], semx0)
    hx1 = pltpu.async_copy(x_hbm.at[pl.ds(base + half, half)],
                           x_full.at[pl.ds(half, half)], semx1)

    def start(kk):
      sl = kk % 2
      return pltpu.async_copy(cen_hbm.at[idx_v.at[pl.ds(kk * CB, CB)]],
                              cbufs[sl], semc[sl])

    pltpu.sync_copy(lab_hbm.at[pl.ds(base, CB)], idx_v.at[pl.ds(0, CB)])
    handles = [start(0), None]
    pltpu.sync_copy(lab_hbm.at[pl.ds(base + CB, BPW - CB)],
                    idx_v.at[pl.ds(CB, BPW - CB)])
    handles[1] = start(1)

    rows16 = jnp.arange(L, dtype=jnp.int32) * L

    def chunk_compute(acc, kk):
      cb = cbufs[kk % 2]
      sbase = kk * CB

      def blk_body(b, acc):
        s0 = b * L
        for i in range(L):
          s = s0 + i
          r = None
          for g in range(GROUPS):
            dv = x_full[sbase + s, pl.ds(g * L, L)] - cb[s, pl.ds(g * L, L)]
            sq = dv * dv
            r = sq if r is None else r + sq
          rs_buf[pl.ds(i * L, L)] = r
        dist = jnp.zeros((L,), jnp.float32)
        for col in range(L):
          dist = dist + plsc.load_gather(rs_buf, [rows16 + col])
        dist = jnp.minimum(jnp.maximum(dist, 1e-12), 1e12)
        return acc + dist

      return lax.fori_loop(0, CB // L, blk_body, acc)

    hx0.wait()
    acc = jnp.zeros((L,), jnp.float32)
    for kk in range(CH):
      handles[kk % 2].wait()
      if kk == CH // 2:
        hx1.wait()
      acc = chunk_compute(acc, kk)
      if kk + 2 < CH:
        handles[kk % 2] = start(kk + 2)

    acc_v[...] = acc
    pltpu.sync_copy(acc_v, out_hbm.at[wid])

  return sc_kernel(x, labels, centers)


def _final_mean(partials):
  def body(p_ref, o_ref):
    o_ref[...] = jnp.sum(p_ref[...]).reshape(1, 1) * (1.0 / BATCH)

  return pl.pallas_call(
      body,
      out_shape=jax.ShapeDtypeStruct((1, 1), jnp.float32),
  )(partials)


def kernel(x, labels, centers):
  partials = _sc_partials(x, labels.astype(jnp.int32), centers)
  return _final_mean(partials)[0, 0]
